# baseline (speedup 1.0000x reference)
import functools

import jax
import jax.numpy as jnp
import numpy as np
from jax import lax
from jax.experimental import pallas as pl
from jax.experimental.pallas import tpu as pltpu
from jax.experimental.pallas import tpu_sc as plsc

N = 10000
G = 16
OUT = 2
GF = 7
H = 32

N_PAD = 10240
E_TOT = 330000

NC, NS, L = 2, 16, 16
NW = NC * NS
CHUNK = 128
NCH = 81
EPW = NCH * CHUNK
E_PAD = NW * EPW

NSEG = N_PAD // NW
CAP = 12288
NCHB = CAP // CHUNK
E_CAPT = NW * CAP

BM = 2048

_MESH = dict(core_axis_name="c", subcore_axis_name="s", num_cores=NC,
             num_subcores=NS)
_SC_PARAMS = pltpu.CompilerParams(needs_layout_passes=False)


def _mm_body(h_ref, w_ref, av_ref, hw_ref, as_ref, ad_ref):
    hw = jnp.dot(h_ref[...], w_ref[...], preferred_element_type=jnp.float32)
    hw_ref[...] = hw
    asad = jnp.dot(hw, av_ref[...], preferred_element_type=jnp.float32)
    as_ref[...] = asad[:, 0:1]
    ad_ref[...] = asad[:, 1:2]


def _mm(h, w, av):
    di, do = w.shape
    return pl.pallas_call(
        _mm_body,
        grid=(N_PAD // BM,),
        in_specs=[
            pl.BlockSpec((BM, di), lambda i: (i, 0)),
            pl.BlockSpec((di, do), lambda i: (0, 0)),
            pl.BlockSpec((do, 2), lambda i: (0, 0)),
        ],
        out_specs=[
            pl.BlockSpec((BM, do), lambda i: (i, 0)),
            pl.BlockSpec((BM, 1), lambda i: (i, 0)),
            pl.BlockSpec((BM, 1), lambda i: (i, 0)),
        ],
        out_shape=[
            jax.ShapeDtypeStruct((N_PAD, do), jnp.float32),
            jax.ShapeDtypeStruct((N_PAD, 1), jnp.float32),
            jax.ShapeDtypeStruct((N_PAD, 1), jnp.float32),
        ],
    )(h, w, av)


def _stats_body(o2_ref, g_ref, b_ref, h_ref):
    do = g_ref.shape[1]
    agg = o2_ref[0][:, :do] + o2_ref[1][:, :do]
    rid = lax.broadcasted_iota(jnp.int32, (N_PAD, do), 0)
    m = rid < N
    aggm = jnp.where(m, agg, 0.0)
    mu = jnp.sum(aggm, axis=0, keepdims=True) * (1.0 / N)
    var = jnp.sum(aggm * aggm, axis=0, keepdims=True) * (1.0 / N) - mu * mu
    hn = g_ref[...] * (agg - mu) / jnp.sqrt(var + 1e-5) + b_ref[...]
    h_ref[...] = jnp.where(m, jnp.maximum(hn, 0.0), 0.0)


def _stats(out2, gamma, beta):
    do = gamma.shape[1]
    return pl.pallas_call(
        _stats_body,
        out_shape=jax.ShapeDtypeStruct((N_PAD, do), jnp.float32),
    )(out2, gamma, beta)


def _statsd_body(o2_ref, den_ref, g_ref, b_ref, h_ref):
    do = g_ref.shape[1]
    agg = (o2_ref[0][:, :do] + o2_ref[1][:, :do]) / den_ref[...]
    rid = lax.broadcasted_iota(jnp.int32, (N_PAD, do), 0)
    m = rid < N
    aggm = jnp.where(m, agg, 0.0)
    mu = jnp.sum(aggm, axis=0, keepdims=True) * (1.0 / N)
    var = jnp.sum(aggm * aggm, axis=0, keepdims=True) * (1.0 / N) - mu * mu
    hn = g_ref[...] * (agg - mu) / jnp.sqrt(var + 1e-5) + b_ref[...]
    h_ref[...] = jnp.where(m, jnp.maximum(hn, 0.0), 0.0)


def _statsd(out2, den_col, gamma, beta):
    do = gamma.shape[1]
    return pl.pallas_call(
        _statsd_body,
        out_shape=jax.ShapeDtypeStruct((N_PAD, do), jnp.float32),
    )(out2, den_col, gamma, beta)


def _den_body(epw, nch, srcf_hbm, dstf_hbm, as_hbm, ad_hbm, p_hbm, den32_hbm,
              srcv, dstv, pv, asv, adv, denv, tmpv):
    cid = lax.axis_index("c")
    sid = lax.axis_index("s")
    wid = sid * NC + cid
    base = wid * epw
    pltpu.sync_copy(srcf_hbm.at[pl.ds(base, epw)], srcv)
    pltpu.sync_copy(dstf_hbm.at[pl.ds(base, epw)], dstv)
    pltpu.sync_copy(as_hbm, asv)
    pltpu.sync_copy(ad_hbm, adv)

    zero = jnp.zeros((L,), jnp.float32)

    def z_body(i, acc):
        denv[pl.ds(i * L, L)] = zero
        return jnp.maximum(acc, asv[pl.ds(i * L, L)])

    acc = lax.fori_loop(0, N_PAD // L, z_body,
                        jnp.full((L,), -1e30, jnp.float32))
    lanes = lax.iota(jnp.int32, L)
    for s in (8, 4, 2, 1):
        perm = lax.rem(lanes + s, jnp.int32(L))
        tmpv[pl.ds(0, L)] = acc
        acc = jnp.maximum(acc, plsc.load_gather(tmpv, [perm]))
    amax = acc

    def chunk(j, carry):
        for k in range(CHUNK // L):
            off = j * CHUNK + k * L
            si = srcv[pl.ds(off, L)]
            di_ = dstv[pl.ds(off, L)]
            asg = plsc.load_gather(asv, [si])
            adg = plsc.load_gather(adv, [di_])
            e = asg + adg
            e = jnp.maximum(e, 0.2 * e)
            cc = amax + adg
            cc = jnp.maximum(cc, 0.2 * cc)
            pp = jnp.exp(e - cc)
            pv[pl.ds(off, L)] = pp
            plsc.addupdate_scatter(denv, [di_], pp)
        return carry

    lax.fori_loop(0, nch, chunk, 0)
    pltpu.sync_copy(pv, p_hbm.at[pl.ds(base, epw)])
    pltpu.sync_copy(denv, den32_hbm.at[pl.ds(wid * N_PAD, N_PAD)])


def _sc_den(srcf, dstf, a_s, a_d, epw, nch):
    etot = NW * epw
    return pl.kernel(
        functools.partial(_den_body, epw, nch),
        out_type=[
            jax.ShapeDtypeStruct((etot,), jnp.float32),
            jax.ShapeDtypeStruct((NW * N_PAD,), jnp.float32),
        ],
        mesh=plsc.VectorSubcoreMesh(**_MESH),
        compiler_params=_SC_PARAMS,
        scratch_types=[
            pltpu.VMEM((epw,), jnp.int32),
            pltpu.VMEM((epw,), jnp.int32),
            pltpu.VMEM((epw,), jnp.float32),
            pltpu.VMEM((N_PAD,), jnp.float32),
            pltpu.VMEM((N_PAD,), jnp.float32),
            pltpu.VMEM((N_PAD,), jnp.float32),
            pltpu.VMEM((L,), jnp.float32),
        ],
    )(srcf, dstf, a_s, a_d)


def _dencomb_body(d32_ref, den_ref):
    den_ref[...] = jnp.sum(d32_ref[...], axis=0, keepdims=True)


def _dencomb(den32):
    return pl.pallas_call(
        _dencomb_body,
        out_shape=jax.ShapeDtypeStruct((1, N_PAD), jnp.float32),
    )(den32)


DW = 128


def _agg_body(dreal, ch, nch, src3_hbm, dst3_hbm, p3_hbm, den_hbm,
              hw_hbm, out2_hbm, idxc, dstc, pc, denv, rows0, rows1, alphav,
              sout, ssem0, ssem1, gsem0, gsem1):
    rows = (rows0, rows1)
    ssem = (ssem0, ssem1)
    gsem = (gsem0, gsem1)
    nv = dreal // L

    cid = lax.axis_index("c")
    sid = lax.axis_index("s")
    wid = sid * NC + cid
    pltpu.sync_copy(den_hbm, denv)

    zero = jnp.zeros((L,), jnp.float32)

    def zrows(r, c):
        for v in range(DW // L):
            rows0[r, pl.ds(v * L, L)] = zero
        return c

    lax.fori_loop(0, ch, zrows, 0)

    seg = N_PAD // NS
    sizes = [ch] * (seg // ch) + ([seg % ch] if seg % ch else [])
    off = 0
    for sz in sizes:
        pltpu.sync_copy(rows0.at[pl.ds(0, sz)],
                        sout.at[pl.ds(sid * seg + off, sz)])
        off += sz
    plsc.subcore_barrier()

    def stage(b, c):
        eb = wid * EPW + c * ch
        pltpu.async_copy(src3_hbm.at[pl.ds(eb, ch)], idxc.at[b], ssem[b])
        pltpu.async_copy(dst3_hbm.at[pl.ds(eb, ch)], dstc.at[b], ssem[b])
        pltpu.async_copy(p3_hbm.at[pl.ds(eb, ch)], pc.at[b], ssem[b])

    def stage_wait(b, c):
        eb = wid * EPW + c * ch
        pltpu.make_async_copy(src3_hbm.at[pl.ds(eb, ch)], idxc.at[b],
                              ssem[b]).wait()
        pltpu.make_async_copy(dst3_hbm.at[pl.ds(eb, ch)], dstc.at[b],
                              ssem[b]).wait()
        pltpu.make_async_copy(p3_hbm.at[pl.ds(eb, ch)], pc.at[b],
                              ssem[b]).wait()

    def process(b, c):
        pltpu.make_async_copy(hw_hbm.at[idxc.at[b]], rows[b], gsem[b]).wait()
        for k in range(ch // L):
            dix = dstc[b, pl.ds(k * L, L)]
            pp = pc[b, pl.ds(k * L, L)]
            deng = plsc.load_gather(denv, [dix])
            al = pp / deng
            alphav[pl.ds(k * L, L)] = al

        def edge_body(e, carry):
            asp = plsc.load_gather(alphav, [jnp.full((L,), e, jnp.int32)])
            for v in range(nv):
                sl = pl.ds(v * L, L)
                rows[b][e, sl] = rows[b][e, sl] * asp
            return carry

        lax.fori_loop(0, ch, edge_body, 0)
        pltpu.sync_copy(rows[b], sout.at[dstc.at[b]], add=True)

    stage(0, 0)
    stage(1, 1)
    npair = nch // 2

    def pair(jj, carry):
        c0 = 2 * jj
        stage_wait(0, c0)
        pltpu.async_copy(hw_hbm.at[idxc.at[0]], rows0, gsem0)
        stage_wait(1, c0 + 1)
        pltpu.async_copy(hw_hbm.at[idxc.at[1]], rows1, gsem1)
        process(0, c0)
        stage(0, c0 + 2)
        process(1, c0 + 1)

        @pl.when(jj < npair - 1)
        def _():
            stage(1, c0 + 3)

        return carry

    lax.fori_loop(0, npair, pair, 0)
    clast = nch - 1
    stage_wait(0, clast)
    pltpu.async_copy(hw_hbm.at[idxc.at[0]], rows0, gsem0)
    process(0, clast)

    plsc.subcore_barrier()
    off = 0
    for sz in sizes:
        r0 = sid * seg + off
        pltpu.sync_copy(sout.at[pl.ds(r0, sz)],
                        out2_hbm.at[cid, pl.ds(r0, sz)])
        off += sz


def _sc_agg(src3, dst3, p3, den, hw, dreal):
    ch, nch = 128, 81
    scratch = [
        pltpu.VMEM((2, ch), jnp.int32),
        pltpu.VMEM((2, ch), jnp.int32),
        pltpu.VMEM((2, ch), jnp.float32),
        pltpu.VMEM((N_PAD,), jnp.float32),
        pltpu.VMEM((ch, DW), jnp.float32),
        pltpu.VMEM((ch, DW), jnp.float32),
        pltpu.VMEM((ch,), jnp.float32),
        pltpu.VMEM_SHARED((N_PAD, DW), jnp.float32),
        pltpu.SemaphoreType.DMA,
        pltpu.SemaphoreType.DMA,
        pltpu.SemaphoreType.DMA,
        pltpu.SemaphoreType.DMA,
    ]
    body = functools.partial(_agg_body, dreal, ch, nch)
    return pl.kernel(
        body,
        out_type=[jax.ShapeDtypeStruct((NC, N_PAD, DW), jnp.float32)],
        mesh=plsc.VectorSubcoreMesh(**_MESH),
        compiler_params=_SC_PARAMS,
        scratch_types=scratch,
    )(src3, dst3, p3, den, hw)


def _srcagg_body(do, with_imp, srcp_hbm, dstp_hbm, p_hbm, hwf_hbm, *rest):
    if with_imp:
        (out2_hbm, imp2_hbm, hws, srcc, dstc, pc, riv, ob, impv,
         sout, ssem0, ssem1) = rest
    else:
        (out2_hbm, hws, srcc, dstc, pc, riv, ob, impv,
         sout, ssem0, ssem1) = rest
        imp2_hbm = None
    ssem = (ssem0, ssem1)
    nvec = do // L

    cid = lax.axis_index("c")
    sid = lax.axis_index("s")
    wid = sid * NC + cid
    base_n = wid * NSEG
    pltpu.sync_copy(hwf_hbm.at[pl.ds(base_n * do, NSEG * do)], hws)

    zero = jnp.zeros((L,), jnp.float32)

    def zrows(r, c):
        for v in range(DW // L):
            ob[r, pl.ds(v * L, L)] = zero
        return c

    lax.fori_loop(0, CHUNK, zrows, 0)
    seg = N_PAD // NS
    for i in range(seg // CHUNK):
        pltpu.sync_copy(ob,
                        sout.at[pl.ds(sid * seg + i * CHUNK, CHUNK)])
    if with_imp:
        def zimp(i, c):
            impv[pl.ds(i * L, L)] = zero
            return c

        lax.fori_loop(0, N_PAD // L, zimp, 0)
    plsc.subcore_barrier()

    lanes = lax.iota(jnp.int32, L)

    def stage(b, c):
        eb = wid * CAP + c * CHUNK
        pltpu.async_copy(srcp_hbm.at[pl.ds(eb, CHUNK)], srcc.at[b], ssem[b])
        pltpu.async_copy(dstp_hbm.at[pl.ds(eb, CHUNK)], dstc.at[b], ssem[b])
        pltpu.async_copy(p_hbm.at[pl.ds(eb, CHUNK)], pc.at[b], ssem[b])

    def stage_wait(b, c):
        eb = wid * CAP + c * CHUNK
        pltpu.make_async_copy(srcp_hbm.at[pl.ds(eb, CHUNK)], srcc.at[b],
                              ssem[b]).wait()
        pltpu.make_async_copy(dstp_hbm.at[pl.ds(eb, CHUNK)], dstc.at[b],
                              ssem[b]).wait()
        pltpu.make_async_copy(p_hbm.at[pl.ds(eb, CHUNK)], pc.at[b],
                              ssem[b]).wait()

    colv = [lanes + v * L for v in range(nvec)]

    def process(b):
        for k in range(CHUNK // L):
            sl = pl.ds(k * L, L)
            sv = srcc[b, sl]
            dv = dstc[b, sl]
            pp = pc[b, sl]
            rv = (sv - base_n) * do
            if with_imp:
                plsc.addupdate_scatter(impv, [dv],
                                       jnp.where(sv != dv, pp, 0.0))
            for e16 in range(L):
                ef = jnp.full((L,), e16, jnp.int32)
                rsp = rv.at[ef].get(mode="promise_in_bounds")
                psp = pp.at[ef].get(mode="promise_in_bounds")
                e = k * L + e16
                for v in range(nvec):
                    g = plsc.load_gather(hws, [rsp + colv[v]])
                    ob[e, pl.ds(v * L, L)] = g * psp
        pltpu.sync_copy(ob, sout.at[dstc.at[b]], add=True)

    stage(0, 0)
    stage(1, 1)
    npair = NCHB // 2

    def pair(jj, carry):
        c0 = 2 * jj
        stage_wait(0, c0)
        process(0)

        @pl.when(jj < npair - 1)
        def _():
            stage(0, c0 + 2)

        stage_wait(1, c0 + 1)
        process(1)

        @pl.when(jj < npair - 1)
        def _():
            stage(1, c0 + 3)

        return carry

    lax.fori_loop(0, npair, pair, 0)

    if with_imp:
        pltpu.sync_copy(impv, imp2_hbm.at[pl.ds(wid * N_PAD, N_PAD)])
    plsc.subcore_barrier()
    for i in range(seg // CHUNK):
        r0 = sid * seg + i * CHUNK
        pltpu.sync_copy(sout.at[pl.ds(r0, CHUNK)],
                        out2_hbm.at[cid, pl.ds(r0, CHUNK)])


def _sc_agg_src(srcp, dstp, p3, hwf, do, with_imp):
    out_type = [jax.ShapeDtypeStruct((NC, N_PAD, DW), jnp.float32)]
    if with_imp:
        out_type.append(jax.ShapeDtypeStruct((NW * N_PAD,), jnp.float32))
    scratch = [
        pltpu.VMEM((NSEG * do,), jnp.float32),
        pltpu.VMEM((2, CHUNK), jnp.int32),
        pltpu.VMEM((2, CHUNK), jnp.int32),
        pltpu.VMEM((2, CHUNK), jnp.float32),
        pltpu.VMEM((CHUNK,), jnp.int32),
        pltpu.VMEM((CHUNK, DW), jnp.float32),
        pltpu.VMEM((N_PAD if with_imp else L,), jnp.float32),
        pltpu.VMEM_SHARED((N_PAD, DW), jnp.float32),
        pltpu.SemaphoreType.DMA,
        pltpu.SemaphoreType.DMA,
    ]
    body = functools.partial(_srcagg_body, do, with_imp)
    return pl.kernel(
        body,
        out_type=out_type,
        mesh=plsc.VectorSubcoreMesh(**_MESH),
        compiler_params=_SC_PARAMS,
        scratch_types=scratch,
    )(srcp, dstp, p3, hwf)


def _tail_body(za_ref, zb_ref, zt_ref, bt_ref, gf_ref,
               wv0, wv1, wv2, wv3, wv4, wv5,
               ma0, ma1, ma2, mba, mb0, mb1, mb2, mbb, mt0, mt1, mt2, mbt,
               bw_ref, w1a, w1b, b1, w2, b2, w3, b3, w4, b4,
               i2a, i2b, i2t, dena, denb, dent,
               h_ref, ia_ref, ib_ref, it_ref):
    f32 = jnp.float32
    bt = bt_ref[...]
    gi = lax.broadcasted_iota(jnp.int32, (G, N_PAD), 0)
    oh = (gi == bt).astype(f32)
    cnt = jnp.sum(oh, axis=1, keepdims=True)
    inv = 1.0 / jnp.maximum(cnt, 1.0)
    oa = jnp.dot(oh, za_ref[...], preferred_element_type=f32) * inv
    ob = jnp.dot(oh, zb_ref[...], preferred_element_type=f32) * inv
    ot = jnp.dot(oh, zt_ref[...], preferred_element_type=f32) * inv

    def mm(a, w):
        return jnp.dot(a, w[...], preferred_element_type=f32)

    fa = jnp.maximum(mm(oa, ma0) + mm(mm(ob, wv0), ma1) +
                     mm(mm(ot, wv1), ma2) + mba[...], 0.0)
    fb = jnp.maximum(mm(ob, mb0) + mm(mm(oa, wv2), mb1) +
                     mm(mm(ot, wv3), mb2) + mbb[...], 0.0)
    ft = jnp.maximum(mm(ot, mt0) + mm(mm(oa, wv4), mt1) +
                     mm(mm(ob, wv5), mt2) + mbt[...], 0.0)

    bw = bw_ref[...]
    bwm = jnp.max(bw, axis=1, keepdims=True)
    ew = jnp.exp(bw - bwm)
    w = ew / jnp.sum(ew, axis=1, keepdims=True)
    fused = w[0, 0] * fa + w[0, 1] * fb + w[0, 2] * ft

    h = jnp.maximum(mm(fused, w1a) + mm(gf_ref[...], w1b) + b1[...], 0.0)
    h = jnp.maximum(mm(h, w2) + b2[...], 0.0)
    h = jnp.maximum(mm(h, w3) + b3[...], 0.0)
    h_ref[...] = mm(h, w4) + b4[...]

    ia_ref[...] = jnp.sum(i2a[...], axis=0, keepdims=True) / dena[...]
    ib_ref[...] = jnp.sum(i2b[...], axis=0, keepdims=True) / denb[...]
    it_ref[...] = jnp.sum(i2t[...], axis=0, keepdims=True) / dent[...]


def _tail(za, zb, zt, bt, gf, attn, mlps, bw, cls, i2a, i2b, i2t, dens):
    wv = [a['Wv'] for a in attn]
    margs = []
    for m in mlps:
        W = m['W']
        margs += [W[0:H], W[H:2 * H], W[2 * H:3 * H], m['b'].reshape(1, H)]
    w1 = cls[0]['W']
    cargs = [w1[0:H], w1[H:H + GF], cls[0]['b'].reshape(1, -1),
             cls[1]['W'], cls[1]['b'].reshape(1, -1),
             cls[2]['W'], cls[2]['b'].reshape(1, -1),
             cls[3]['W'], cls[3]['b'].reshape(1, -1)]
    out_shape = [
        jax.ShapeDtypeStruct((G, OUT), jnp.float32),
        jax.ShapeDtypeStruct((1, N_PAD), jnp.float32),
        jax.ShapeDtypeStruct((1, N_PAD), jnp.float32),
        jax.ShapeDtypeStruct((1, N_PAD), jnp.float32),
    ]
    return pl.pallas_call(_tail_body, out_shape=out_shape)(
        za, zb, zt, bt, gf, *wv, *margs, bw.reshape(1, 3), *cargs,
        i2a, i2b, i2t, dens[0], dens[1], dens[2])


def kernel(x, edge_index, batch, global_feature, params):
    f32 = jnp.float32
    i32 = jnp.int32
    loop = jnp.arange(N, dtype=i32)
    src0 = jnp.concatenate([edge_index[0].astype(i32), loop])
    dst0 = jnp.concatenate([edge_index[1].astype(i32), loop])
    padlen = E_PAD - E_TOT
    fill = jnp.full((padlen,), N, i32)
    src = jnp.concatenate([src0, fill])
    dst = jnp.concatenate([dst0, fill])

    bucket = src0 // NSEG
    onehot = (bucket[:, None] == jnp.arange(NW, dtype=i32)).astype(i32)
    rank = jnp.take_along_axis(jnp.cumsum(onehot, axis=0),
                               bucket[:, None], axis=1)[:, 0] - 1
    pos = bucket * CAP + rank
    srcp = jnp.repeat(jnp.arange(NW, dtype=i32) * NSEG,
                      CAP).at[pos].set(src0)
    dstp = jnp.full((E_CAPT,), N, i32).at[pos].set(dst0)

    x_pad = jnp.zeros((N_PAD, x.shape[1]), f32).at[:N].set(x)
    bt = jnp.full((1, N_PAD), G, i32).at[0, :N].set(batch.astype(i32))

    imps = []
    zs = []
    dens = []
    for band in params['bands']:
        h = x_pad
        imp2 = None
        den_row = None
        for li, p in enumerate(band):
            di, do = p['W'].shape
            last = li == len(band) - 1
            if li == 0:
                w_pad = jnp.zeros((di, DW), f32).at[:, :do].set(p['W'])
                av = jnp.zeros((DW, 2), f32)
                av = av.at[:do, 0].set(p['a_src']).at[:do, 1].set(p['a_dst'])
                hw, aso, ado = _mm(h, w_pad, av)
                a_s = aso.reshape(N_PAD)
                a_d = ado.reshape(N_PAD)
                p3, den32 = _sc_den(src, dst, a_s, a_d, EPW, NCH)
                den = _dencomb(den32.reshape(NW, N_PAD)).reshape(N_PAD)
                out2, = _sc_agg(src, dst, p3, den, hw, do)
                h = _stats(out2, p['gamma'].reshape(1, -1),
                           p['beta'].reshape(1, -1))
            else:
                av = jnp.zeros((do, 2), f32)
                av = av.at[:, 0].set(p['a_src']).at[:, 1].set(p['a_dst'])
                hw, aso, ado = _mm(h, p['W'], av)
                a_s = aso.reshape(N_PAD)
                a_d = ado.reshape(N_PAD)
                p3, den32 = _sc_den(srcp, dstp, a_s, a_d, CAP, NCHB)
                den_row = _dencomb(den32.reshape(NW, N_PAD))
                hwf = hw.reshape(N_PAD * do)
                res = _sc_agg_src(srcp, dstp, p3, hwf, do, last)
                if last:
                    out2, imp2 = res
                else:
                    out2, = res
                h = _statsd(out2, den_row.reshape(N_PAD, 1),
                            p['gamma'].reshape(1, -1),
                            p['beta'].reshape(1, -1))
        zs.append(h)
        imps.append(imp2)
        dens.append(den_row)

    hh, ia, ib, it = _tail(zs[0], zs[1], zs[2], bt, global_feature,
                           params['attn'], params['mlps'],
                           params['band_weights'], params['cls'],
                           imps[0].reshape(NW, N_PAD),
                           imps[1].reshape(NW, N_PAD),
                           imps[2].reshape(NW, N_PAD), dens)
    return (hh, ia[0, :N], ib[0, :N], it[0, :N])


# speedup vs baseline: 1.9173x; 1.9173x over previous
import functools

import jax
import jax.numpy as jnp
import numpy as np
from jax import lax
from jax.experimental import pallas as pl
from jax.experimental.pallas import tpu as pltpu
from jax.experimental.pallas import tpu_sc as plsc

N = 10000
G = 16
OUT = 2
GF = 7
H = 32

N_PAD = 10240
E_TOT = 330000

NC, NS, L = 2, 16, 16
NW = NC * NS
CHUNK = 128
NCH = 81
EPW = NCH * CHUNK
E_PAD = NW * EPW

BM = 2048

_MESH = dict(core_axis_name="c", subcore_axis_name="s", num_cores=NC,
             num_subcores=NS)
_SC_PARAMS = pltpu.CompilerParams(needs_layout_passes=False)


def _mm_body(h_ref, w_ref, av_ref, hw_ref, as_ref, ad_ref):
    hw = jnp.dot(h_ref[...], w_ref[...], preferred_element_type=jnp.float32)
    hw_ref[...] = hw
    asad = jnp.dot(hw, av_ref[...], preferred_element_type=jnp.float32)
    as_ref[...] = asad[:, 0:1]
    ad_ref[...] = asad[:, 1:2]


def _mm(h, w, av):
    di, do = w.shape
    return pl.pallas_call(
        _mm_body,
        grid=(N_PAD // BM,),
        in_specs=[
            pl.BlockSpec((BM, di), lambda i: (i, 0)),
            pl.BlockSpec((di, do), lambda i: (0, 0)),
            pl.BlockSpec((do, 2), lambda i: (0, 0)),
        ],
        out_specs=[
            pl.BlockSpec((BM, do), lambda i: (i, 0)),
            pl.BlockSpec((BM, 1), lambda i: (i, 0)),
            pl.BlockSpec((BM, 1), lambda i: (i, 0)),
        ],
        out_shape=[
            jax.ShapeDtypeStruct((N_PAD, do), jnp.float32),
            jax.ShapeDtypeStruct((N_PAD, 1), jnp.float32),
            jax.ShapeDtypeStruct((N_PAD, 1), jnp.float32),
        ],
    )(h, w, av)


def _stats_body(o2_ref, g_ref, b_ref, h_ref):
    do = g_ref.shape[1]
    agg = o2_ref[0][:, :do] + o2_ref[1][:, :do]
    rid = lax.broadcasted_iota(jnp.int32, (N_PAD, do), 0)
    m = rid < N
    aggm = jnp.where(m, agg, 0.0)
    mu = jnp.sum(aggm, axis=0, keepdims=True) * (1.0 / N)
    var = jnp.sum(aggm * aggm, axis=0, keepdims=True) * (1.0 / N) - mu * mu
    hn = g_ref[...] * (agg - mu) / jnp.sqrt(var + 1e-5) + b_ref[...]
    h_ref[...] = jnp.where(m, jnp.maximum(hn, 0.0), 0.0)


def _stats(out2, gamma, beta):
    do = gamma.shape[1]
    return pl.pallas_call(
        _stats_body,
        out_shape=jax.ShapeDtypeStruct((N_PAD, do), jnp.float32),
    )(out2, gamma, beta)


def _den_body(srcf_hbm, dstf_hbm, as_hbm, ad_hbm, p_hbm, den32_hbm,
              srcv, dstv, pv, asv, adv, denv, tmpv):
    cid = lax.axis_index("c")
    sid = lax.axis_index("s")
    wid = sid * NC + cid
    base = wid * EPW
    pltpu.sync_copy(srcf_hbm.at[pl.ds(base, EPW)], srcv)
    pltpu.sync_copy(dstf_hbm.at[pl.ds(base, EPW)], dstv)
    pltpu.sync_copy(as_hbm, asv)
    pltpu.sync_copy(ad_hbm, adv)

    zero = jnp.zeros((L,), jnp.float32)

    def z_body(i, acc):
        denv[pl.ds(i * L, L)] = zero
        return jnp.maximum(acc, asv[pl.ds(i * L, L)])

    acc = lax.fori_loop(0, N_PAD // L, z_body,
                        jnp.full((L,), -1e30, jnp.float32))
    lanes = lax.iota(jnp.int32, L)
    for s in (8, 4, 2, 1):
        perm = lax.rem(lanes + s, jnp.int32(L))
        tmpv[pl.ds(0, L)] = acc
        acc = jnp.maximum(acc, plsc.load_gather(tmpv, [perm]))
    amax = acc

    def chunk(j, carry):
        for k in range(CHUNK // L):
            off = j * CHUNK + k * L
            si = srcv[pl.ds(off, L)]
            di_ = dstv[pl.ds(off, L)]
            asg = plsc.load_gather(asv, [si])
            adg = plsc.load_gather(adv, [di_])
            e = asg + adg
            e = jnp.maximum(e, 0.2 * e)
            cc = amax + adg
            cc = jnp.maximum(cc, 0.2 * cc)
            pp = jnp.exp(e - cc)
            pv[pl.ds(off, L)] = pp
            plsc.addupdate_scatter(denv, [di_], pp)
        return carry

    lax.fori_loop(0, NCH, chunk, 0)
    pltpu.sync_copy(pv, p_hbm.at[pl.ds(base, EPW)])
    pltpu.sync_copy(denv, den32_hbm.at[pl.ds(wid * N_PAD, N_PAD)])


def _sc_den(srcf, dstf, a_s, a_d):
    return pl.kernel(
        _den_body,
        out_type=[
            jax.ShapeDtypeStruct((E_PAD,), jnp.float32),
            jax.ShapeDtypeStruct((NW * N_PAD,), jnp.float32),
        ],
        mesh=plsc.VectorSubcoreMesh(**_MESH),
        compiler_params=_SC_PARAMS,
        scratch_types=[
            pltpu.VMEM((EPW,), jnp.int32),
            pltpu.VMEM((EPW,), jnp.int32),
            pltpu.VMEM((EPW,), jnp.float32),
            pltpu.VMEM((N_PAD,), jnp.float32),
            pltpu.VMEM((N_PAD,), jnp.float32),
            pltpu.VMEM((N_PAD,), jnp.float32),
            pltpu.VMEM((L,), jnp.float32),
        ],
    )(srcf, dstf, a_s, a_d)


def _dencomb_body(d32_ref, den_ref):
    den_ref[...] = jnp.sum(d32_ref[...], axis=0, keepdims=True)


def _dencomb(den32):
    return pl.pallas_call(
        _dencomb_body,
        out_shape=jax.ShapeDtypeStruct((1, N_PAD), jnp.float32),
    )(den32)


DW = 128


def _agg_body(dreal, ch, nch, with_imp, src3_hbm, dst3_hbm, p3_hbm, den_hbm,
              hw_hbm, *rest):
    if with_imp:
        (out2_hbm, imp2_hbm, idxc, dstc, pc, denv, rows0, rows1, alphav,
         impv, sout, ssem0, ssem1, gsem0, gsem1) = rest
    else:
        (out2_hbm, idxc, dstc, pc, denv, rows0, rows1, alphav,
         impv, sout, ssem0, ssem1, gsem0, gsem1) = rest
        imp2_hbm = None
    rows = (rows0, rows1)
    ssem = (ssem0, ssem1)
    gsem = (gsem0, gsem1)
    nv = dreal // L

    cid = lax.axis_index("c")
    sid = lax.axis_index("s")
    wid = sid * NC + cid
    pltpu.sync_copy(den_hbm, denv)

    zero = jnp.zeros((L,), jnp.float32)

    def zrows(r, c):
        for v in range(DW // L):
            rows0[r, pl.ds(v * L, L)] = zero
        return c

    lax.fori_loop(0, ch, zrows, 0)
    if with_imp:
        def zimp(i, c):
            impv[pl.ds(i * L, L)] = zero
            return c

        lax.fori_loop(0, N_PAD // L, zimp, 0)

    seg = N_PAD // NS
    sizes = [ch] * (seg // ch) + ([seg % ch] if seg % ch else [])
    off = 0
    for sz in sizes:
        pltpu.sync_copy(rows0.at[pl.ds(0, sz)],
                        sout.at[pl.ds(sid * seg + off, sz)])
        off += sz
    plsc.subcore_barrier()

    def stage(b, c):
        eb = wid * EPW + c * ch
        pltpu.async_copy(src3_hbm.at[pl.ds(eb, ch)], idxc.at[b], ssem[b])
        pltpu.async_copy(dst3_hbm.at[pl.ds(eb, ch)], dstc.at[b], ssem[b])
        pltpu.async_copy(p3_hbm.at[pl.ds(eb, ch)], pc.at[b], ssem[b])

    def stage_wait(b, c):
        eb = wid * EPW + c * ch
        pltpu.make_async_copy(src3_hbm.at[pl.ds(eb, ch)], idxc.at[b],
                              ssem[b]).wait()
        pltpu.make_async_copy(dst3_hbm.at[pl.ds(eb, ch)], dstc.at[b],
                              ssem[b]).wait()
        pltpu.make_async_copy(p3_hbm.at[pl.ds(eb, ch)], pc.at[b],
                              ssem[b]).wait()

    def process(b, c):
        pltpu.make_async_copy(hw_hbm.at[idxc.at[b]], rows[b], gsem[b]).wait()
        for k in range(ch // L):
            dix = dstc[b, pl.ds(k * L, L)]
            pp = pc[b, pl.ds(k * L, L)]
            deng = plsc.load_gather(denv, [dix])
            al = pp / deng
            alphav[pl.ds(k * L, L)] = al
            if with_imp:
                six = idxc[b, pl.ds(k * L, L)]
                plsc.addupdate_scatter(
                    impv, [dix], jnp.where(six != dix, al, 0.0))

        def edge_body(e, carry):
            asp = plsc.load_gather(alphav, [jnp.full((L,), e, jnp.int32)])
            for v in range(nv):
                sl = pl.ds(v * L, L)
                rows[b][e, sl] = rows[b][e, sl] * asp
            return carry

        lax.fori_loop(0, ch, edge_body, 0)
        pltpu.sync_copy(rows[b], sout.at[dstc.at[b]], add=True)

    stage(0, 0)
    stage(1, 1)
    npair = nch // 2
    odd = nch % 2 == 1

    def pair(jj, carry):
        c0 = 2 * jj
        stage_wait(0, c0)
        pltpu.async_copy(hw_hbm.at[idxc.at[0]], rows0, gsem0)
        stage_wait(1, c0 + 1)
        pltpu.async_copy(hw_hbm.at[idxc.at[1]], rows1, gsem1)
        process(0, c0)
        if odd:
            stage(0, c0 + 2)
        else:
            @pl.when(jj < npair - 1)
            def _():
                stage(0, c0 + 2)
        process(1, c0 + 1)

        @pl.when(jj < npair - 1)
        def _():
            stage(1, c0 + 3)

        return carry

    lax.fori_loop(0, npair, pair, 0)
    if odd:
        clast = nch - 1
        stage_wait(0, clast)
        pltpu.async_copy(hw_hbm.at[idxc.at[0]], rows0, gsem0)
        process(0, clast)

    if with_imp:
        pltpu.sync_copy(impv, imp2_hbm.at[pl.ds(wid * N_PAD, N_PAD)])
    plsc.subcore_barrier()
    off = 0
    for sz in sizes:
        r0 = sid * seg + off
        pltpu.sync_copy(sout.at[pl.ds(r0, sz)],
                        out2_hbm.at[cid, pl.ds(r0, sz)])
        off += sz


def _sc_agg(src3, dst3, p3, den, hw, dreal, with_imp):
    ch, nch = (96, 108) if with_imp else (128, 81)
    out_type = [jax.ShapeDtypeStruct((NC, N_PAD, DW), jnp.float32)]
    if with_imp:
        out_type.append(jax.ShapeDtypeStruct((NW * N_PAD,), jnp.float32))
    scratch = [
        pltpu.VMEM((2, ch), jnp.int32),
        pltpu.VMEM((2, ch), jnp.int32),
        pltpu.VMEM((2, ch), jnp.float32),
        pltpu.VMEM((N_PAD,), jnp.float32),
        pltpu.VMEM((ch, DW), jnp.float32),
        pltpu.VMEM((ch, DW), jnp.float32),
        pltpu.VMEM((ch,), jnp.float32),
        pltpu.VMEM((N_PAD if with_imp else L,), jnp.float32),
        pltpu.VMEM_SHARED((N_PAD, DW), jnp.float32),
        pltpu.SemaphoreType.DMA,
        pltpu.SemaphoreType.DMA,
        pltpu.SemaphoreType.DMA,
        pltpu.SemaphoreType.DMA,
    ]
    body = functools.partial(_agg_body, dreal, ch, nch, with_imp)
    return pl.kernel(
        body,
        out_type=out_type,
        mesh=plsc.VectorSubcoreMesh(**_MESH),
        compiler_params=_SC_PARAMS,
        scratch_types=scratch,
    )(src3, dst3, p3, den, hw)


def _tail_body(za_ref, zb_ref, zt_ref, bt_ref, gf_ref,
               wv0, wv1, wv2, wv3, wv4, wv5,
               ma0, ma1, ma2, mba, mb0, mb1, mb2, mbb, mt0, mt1, mt2, mbt,
               bw_ref, w1a, w1b, b1, w2, b2, w3, b3, w4, b4,
               i2a, i2b, i2t,
               h_ref, ia_ref, ib_ref, it_ref):
    f32 = jnp.float32
    bt = bt_ref[...]
    gi = lax.broadcasted_iota(jnp.int32, (G, N_PAD), 0)
    oh = (gi == bt).astype(f32)
    cnt = jnp.sum(oh, axis=1, keepdims=True)
    inv = 1.0 / jnp.maximum(cnt, 1.0)
    oa = jnp.dot(oh, za_ref[...], preferred_element_type=f32) * inv
    ob = jnp.dot(oh, zb_ref[...], preferred_element_type=f32) * inv
    ot = jnp.dot(oh, zt_ref[...], preferred_element_type=f32) * inv

    def mm(a, w):
        return jnp.dot(a, w[...], preferred_element_type=f32)

    fa = jnp.maximum(mm(oa, ma0) + mm(mm(ob, wv0), ma1) +
                     mm(mm(ot, wv1), ma2) + mba[...], 0.0)
    fb = jnp.maximum(mm(ob, mb0) + mm(mm(oa, wv2), mb1) +
                     mm(mm(ot, wv3), mb2) + mbb[...], 0.0)
    ft = jnp.maximum(mm(ot, mt0) + mm(mm(oa, wv4), mt1) +
                     mm(mm(ob, wv5), mt2) + mbt[...], 0.0)

    bw = bw_ref[...]
    bwm = jnp.max(bw, axis=1, keepdims=True)
    ew = jnp.exp(bw - bwm)
    w = ew / jnp.sum(ew, axis=1, keepdims=True)
    fused = w[0, 0] * fa + w[0, 1] * fb + w[0, 2] * ft

    h = jnp.maximum(mm(fused, w1a) + mm(gf_ref[...], w1b) + b1[...], 0.0)
    h = jnp.maximum(mm(h, w2) + b2[...], 0.0)
    h = jnp.maximum(mm(h, w3) + b3[...], 0.0)
    h_ref[...] = mm(h, w4) + b4[...]

    ia_ref[...] = jnp.sum(i2a[...], axis=0, keepdims=True)
    ib_ref[...] = jnp.sum(i2b[...], axis=0, keepdims=True)
    it_ref[...] = jnp.sum(i2t[...], axis=0, keepdims=True)


def _tail(za, zb, zt, bt, gf, attn, mlps, bw, cls, i2a, i2b, i2t):
    wv = [a['Wv'] for a in attn]
    margs = []
    for m in mlps:
        W = m['W']
        margs += [W[0:H], W[H:2 * H], W[2 * H:3 * H], m['b'].reshape(1, H)]
    w1 = cls[0]['W']
    cargs = [w1[0:H], w1[H:H + GF], cls[0]['b'].reshape(1, -1),
             cls[1]['W'], cls[1]['b'].reshape(1, -1),
             cls[2]['W'], cls[2]['b'].reshape(1, -1),
             cls[3]['W'], cls[3]['b'].reshape(1, -1)]
    out_shape = [
        jax.ShapeDtypeStruct((G, OUT), jnp.float32),
        jax.ShapeDtypeStruct((1, N_PAD), jnp.float32),
        jax.ShapeDtypeStruct((1, N_PAD), jnp.float32),
        jax.ShapeDtypeStruct((1, N_PAD), jnp.float32),
    ]
    return pl.pallas_call(_tail_body, out_shape=out_shape)(
        za, zb, zt, bt, gf, *wv, *margs, bw.reshape(1, 3), *cargs,
        i2a, i2b, i2t)


def kernel(x, edge_index, batch, global_feature, params):
    f32 = jnp.float32
    loop = jnp.arange(N, dtype=jnp.int32)
    src = jnp.concatenate([edge_index[0].astype(jnp.int32), loop])
    dst = jnp.concatenate([edge_index[1].astype(jnp.int32), loop])
    padlen = E_PAD - E_TOT
    fill = jnp.full((padlen,), N, jnp.int32)
    src = jnp.concatenate([src, fill])
    dst = jnp.concatenate([dst, fill])

    x_pad = jnp.zeros((N_PAD, x.shape[1]), f32).at[:N].set(x)
    bt = jnp.full((1, N_PAD), G, jnp.int32).at[0, :N].set(
        batch.astype(jnp.int32))

    imps = []
    zs = []
    for band in params['bands']:
        h = x_pad
        imp2 = None
        for li, p in enumerate(band):
            di, do = p['W'].shape
            w_pad = jnp.zeros((di, DW), f32).at[:, :do].set(p['W'])
            av = jnp.zeros((DW, 2), f32)
            av = av.at[:do, 0].set(p['a_src']).at[:do, 1].set(p['a_dst'])
            hw, aso, ado = _mm(h, w_pad, av)
            a_s = aso.reshape(N_PAD)
            a_d = ado.reshape(N_PAD)
            p3, den32 = _sc_den(src, dst, a_s, a_d)
            den = _dencomb(den32.reshape(NW, N_PAD)).reshape(N_PAD)
            last = li == len(band) - 1
            res = _sc_agg(src, dst, p3, den, hw, do, last)
            if last:
                out2, imp2 = res
            else:
                out2, = res
            h = _stats(out2, p['gamma'].reshape(1, -1),
                       p['beta'].reshape(1, -1))
        zs.append(h)
        imps.append(imp2)

    hh, ia, ib, it = _tail(zs[0], zs[1], zs[2], bt, global_feature,
                           params['attn'], params['mlps'],
                           params['band_weights'], params['cls'],
                           imps[0].reshape(NW, N_PAD),
                           imps[1].reshape(NW, N_PAD),
                           imps[2].reshape(NW, N_PAD))
    return (hh, ia[0, :N], ib[0, :N], it[0, :N])
